# baseline (speedup 1.0000x reference)
import functools
import jax
import jax.numpy as jnp
from jax import lax
from jax.experimental import pallas as pl
from jax.experimental.pallas import tpu as pltpu
from jax.experimental.pallas import tpu_sc as plsc

_FBLK = 32


def _sc_lookup(seq, feature, table_dtype):
    info = plsc.get_sparse_core_info()
    nc, ns, lanes = info.num_cores, info.num_subcores, info.num_lanes
    nw = nc * ns
    rows_per_w = seq // nw
    mesh = plsc.VectorSubcoreMesh(core_axis_name="c", subcore_axis_name="s")

    @functools.partial(
        pl.kernel,
        mesh=mesh,
        out_type=jax.ShapeDtypeStruct((seq, feature), table_dtype),
        scratch_types=[
            pltpu.VMEM((rows_per_w,), jnp.int32),
            pltpu.VMEM((rows_per_w, feature), table_dtype),
            pltpu.SemaphoreType.DMA,
        ],
    )
    def lookup(table_hbm, out_hbm, idx_v, rows_v, sem):
        wid = lax.axis_index("s") * nc + lax.axis_index("c")
        base = wid * rows_per_w
        for j in range(rows_per_w // lanes):
            idx_v[pl.ds(j * lanes, lanes)] = (
                lax.iota(jnp.int32, lanes) + base + j * lanes
            )
        pltpu.async_copy(table_hbm.at[idx_v], rows_v, sem).wait()
        pltpu.sync_copy(rows_v, out_hbm.at[pl.ds(base, rows_per_w)])

    return lookup


def _pe_add_kernel(inp_ref, pos_ref, out_ref, pos_t):
    b = pl.program_id(0)
    g = pl.program_id(1)

    @pl.when(jnp.logical_and(b == 0, g == 0))
    def _():
        pos_t[...] = pos_ref[...].T

    slab = pos_t[pl.ds(g * _FBLK, _FBLK), :]
    out_ref[...] = inp_ref[...] + slab[None, :, None, :]


def kernel(input_tensor, pos_table):
    batch, feature, height, seq = input_tensor.shape

    pos_rows = _sc_lookup(seq, feature, pos_table.dtype)(pos_table)

    grid = (batch, feature // _FBLK)
    return pl.pallas_call(
        _pe_add_kernel,
        grid=grid,
        in_specs=[
            pl.BlockSpec((1, _FBLK, height, seq), lambda b, g: (b, g, 0, 0)),
            pl.BlockSpec((seq, feature), lambda b, g: (0, 0)),
        ],
        out_specs=pl.BlockSpec((1, _FBLK, height, seq), lambda b, g: (b, g, 0, 0)),
        out_shape=jax.ShapeDtypeStruct(input_tensor.shape, input_tensor.dtype),
        scratch_shapes=[pltpu.VMEM((feature, seq), input_tensor.dtype)],
    )(input_tensor, pos_rows)


# speedup vs baseline: 1.0264x; 1.0264x over previous
import functools
import jax
import jax.numpy as jnp
from jax import lax
from jax.experimental import pallas as pl
from jax.experimental.pallas import tpu as pltpu
from jax.experimental.pallas import tpu_sc as plsc

_FBLK = 32


def _sc_lookup(seq, feature, table_dtype):
    info = plsc.get_sparse_core_info()
    nc, ns, lanes = info.num_cores, info.num_subcores, info.num_lanes
    nw = nc * ns
    rows_per_w = seq // nw
    mesh = plsc.VectorSubcoreMesh(core_axis_name="c", subcore_axis_name="s")

    @functools.partial(
        pl.kernel,
        mesh=mesh,
        out_type=jax.ShapeDtypeStruct((seq, feature), table_dtype),
        scratch_types=[
            pltpu.VMEM((rows_per_w,), jnp.int32),
            pltpu.VMEM((rows_per_w, feature), table_dtype),
            pltpu.SemaphoreType.DMA,
        ],
    )
    def lookup(table_hbm, out_hbm, idx_v, rows_v, sem):
        wid = lax.axis_index("s") * nc + lax.axis_index("c")
        base = wid * rows_per_w
        for j in range(rows_per_w // lanes):
            idx_v[pl.ds(j * lanes, lanes)] = (
                lax.iota(jnp.int32, lanes) + base + j * lanes
            )
        pltpu.async_copy(table_hbm.at[idx_v], rows_v, sem).wait()
        pltpu.sync_copy(rows_v, out_hbm.at[pl.ds(base, rows_per_w)])

    return lookup


def _pe_add_a(inp_ref, pos_ref, out_ref, pos_t):
    g = pl.program_id(0)

    @pl.when(g == 0)
    def _():
        pos_t[...] = pos_ref[...].T

    slab = pos_t[pl.ds(g * _FBLK, _FBLK), :]
    out_ref[...] = inp_ref[...] + slab[None, :, None, :]


def _pe_add_b(inp_ref, pos_ref, prev_ref, out_ref, pos_t):
    b = pl.program_id(0)
    g = pl.program_id(1)
    del prev_ref

    @pl.when(jnp.logical_and(b == 0, g == 0))
    def _():
        pos_t[...] = pos_ref[...].T

    slab = pos_t[pl.ds(g * _FBLK, _FBLK), :]
    out_ref[...] = inp_ref[...] + slab[None, :, None, :]


def kernel(input_tensor, pos_table):
    batch, feature, height, seq = input_tensor.shape
    groups = feature // _FBLK
    blk = (1, _FBLK, height, seq)
    full = jax.ShapeDtypeStruct(input_tensor.shape, input_tensor.dtype)

    pos_rows = _sc_lookup(seq, feature, pos_table.dtype)(pos_table)

    out_a = pl.pallas_call(
        _pe_add_a,
        grid=(groups,),
        in_specs=[
            pl.BlockSpec(blk, lambda g: (0, g, 0, 0)),
            pl.BlockSpec((seq, feature), lambda g: (0, 0)),
        ],
        out_specs=pl.BlockSpec(blk, lambda g: (0, g, 0, 0)),
        out_shape=full,
        scratch_shapes=[pltpu.VMEM((feature, seq), input_tensor.dtype)],
    )(input_tensor, pos_table)

    return pl.pallas_call(
        _pe_add_b,
        grid=(batch - 1, groups),
        in_specs=[
            pl.BlockSpec(blk, lambda b, g: (b + 1, g, 0, 0)),
            pl.BlockSpec((seq, feature), lambda b, g: (0, 0)),
            pl.BlockSpec(memory_space=pl.ANY),
        ],
        out_specs=pl.BlockSpec(blk, lambda b, g: (b + 1, g, 0, 0)),
        out_shape=full,
        input_output_aliases={2: 0},
        scratch_shapes=[pltpu.VMEM((feature, seq), input_tensor.dtype)],
    )(input_tensor, pos_rows, out_a)
